# Initial kernel scaffold; baseline (speedup 1.0000x reference)
import jax
import jax.numpy as jnp
from jax.experimental import pallas as pl


def kernel(h, x_s, edge_index, node_mask, edge_mask, W_q, W_k, W_v, W_static, W_out, ln_ds_w, ln_ds_b, msg_W1, msg_b1, msg_W2, msg_b2, gate_W1, gate_b1, gate_W2, gate_b2, ln_w, ln_b, W_comb, b_comb):
    raise NotImplementedError("write your pallas kernel here")


# speedup vs baseline: 1.7641x; 1.7641x over previous
import functools

import jax
import jax.numpy as jnp
import numpy as np
from jax.experimental import pallas as pl
from jax.experimental.pallas import tpu as pltpu

_N = 10000
_E = 160000
_H = 256
_NH = 8
_HD = 32
_S = 16

_BN = 1000
_BE = 2000


def _lnorm(x, w, b):
    m = jnp.mean(x, axis=-1, keepdims=True)
    v = jnp.mean((x - m) ** 2, axis=-1, keepdims=True)
    return (x - m) / jnp.sqrt(v + 1e-5) * w + b


def _dot(a, b):
    return jnp.dot(a, b, preferred_element_type=jnp.float32)


def _node_kernel(h_ref, xs_ref, wq_ref, wk_ref, wv_ref, wstat_ref,
                 lnds_w_ref, lnds_b_ref, m1h_ref, m1x_ref, mb1_ref,
                 m2_ref, mb2_ref, gs_ref, gr_ref, gxs_ref, gxr_ref,
                 gb1_ref, lnw_ref, lnb_ref,
                 q_ref, k_ref, v_ref, pm_ref, ps_ref, pr_ref):
    h = h_ref[...]
    xs = xs_ref[...]
    hn = _lnorm(h, lnw_ref[...], lnb_ref[...])
    hc = _lnorm(hn + _dot(xs, wstat_ref[...]), lnds_w_ref[...], lnds_b_ref[...])
    q_ref[...] = _dot(hc, wq_ref[...])
    k_ref[...] = _dot(hc, wk_ref[...])
    v_ref[...] = _dot(hc, wv_ref[...])
    m1 = jnp.maximum(_dot(hn, m1h_ref[...]) + _dot(xs, m1x_ref[...]) + mb1_ref[...], 0.0)
    pm_ref[...] = _dot(m1, m2_ref[...]) + mb2_ref[...]
    ps_ref[...] = _dot(hn, gs_ref[...]) + _dot(xs, gxs_ref[...]) + gb1_ref[...]
    pr_ref[...] = _dot(hn, gr_ref[...]) + _dot(xs, gxr_ref[...])


def _node_precompute(h, x_s, W_q, W_k, W_v, W_static, ln_ds_w, ln_ds_b,
                     msg_W1, msg_b1, msg_W2, msg_b2, gate_W1, gate_b1,
                     ln_w, ln_b):
    grid = _N // _BN
    row = lambda i: (i, 0)
    full = lambda i: (0, 0)
    hspec = pl.BlockSpec((_BN, _H), row)
    xspec = pl.BlockSpec((_BN, _S), row)
    wspec = pl.BlockSpec((_H, _H), full)
    sspec = pl.BlockSpec((_S, _H), full)
    bspec = pl.BlockSpec((1, _H), full)
    out = [jax.ShapeDtypeStruct((_N, _H), jnp.float32) for _ in range(6)]
    return pl.pallas_call(
        _node_kernel,
        grid=(grid,),
        in_specs=[hspec, xspec, wspec, wspec, wspec, sspec, bspec, bspec,
                  wspec, sspec, bspec, wspec, bspec, wspec, wspec, sspec,
                  sspec, bspec, bspec, bspec],
        out_specs=[hspec] * 6,
        out_shape=out,
    )(h, x_s, W_q, W_k, W_v, W_static,
      ln_ds_w.reshape(1, _H), ln_ds_b.reshape(1, _H),
      msg_W1[:_H], msg_W1[_H:], msg_b1.reshape(1, _H), msg_W2,
      msg_b2.reshape(1, _H),
      gate_W1[:_H], gate_W1[_H:2 * _H], gate_W1[2 * _H:2 * _H + _S],
      gate_W1[2 * _H + _S:], gate_b1.reshape(1, _H),
      ln_w.reshape(1, _H), ln_b.reshape(1, _H))


def _escore_kernel(qd_ref, ks_ref, sel_ref, e_ref):
    prod = qd_ref[...] * ks_ref[...]
    s = _dot(prod, sel_ref[...]) * (1.0 / np.sqrt(float(_HD)))
    e_ref[...] = jnp.exp(s)


def _escore(qd, ks):
    sel = jnp.repeat(jnp.eye(_NH, dtype=jnp.float32), _HD, axis=0)
    grid = _E // _BE
    row = lambda i: (i, 0)
    return pl.pallas_call(
        _escore_kernel,
        grid=(grid,),
        in_specs=[pl.BlockSpec((_BE, _H), row), pl.BlockSpec((_BE, _H), row),
                  pl.BlockSpec((_H, _NH), lambda i: (0, 0))],
        out_specs=pl.BlockSpec((_BE, _NH), row),
        out_shape=jax.ShapeDtypeStruct((_E, _NH), jnp.float32),
    )(qd, ks, sel)


def _attn_kernel(e_ref, zd_ref, vs_ref, exp_ref, attn_ref, wv_ref):
    attn = e_ref[...] / (zd_ref[...] + 1e-9)
    attn_ref[...] = attn
    wv_ref[...] = vs_ref[...] * _dot(attn, exp_ref[...])


def _attn_wv(e, zd, vs):
    expand = jnp.repeat(jnp.eye(_NH, dtype=jnp.float32), _HD, axis=1)
    grid = _E // _BE
    row = lambda i: (i, 0)
    return pl.pallas_call(
        _attn_kernel,
        grid=(grid,),
        in_specs=[pl.BlockSpec((_BE, _NH), row), pl.BlockSpec((_BE, _NH), row),
                  pl.BlockSpec((_BE, _H), row),
                  pl.BlockSpec((_NH, _H), lambda i: (0, 0))],
        out_specs=[pl.BlockSpec((_BE, _NH), row), pl.BlockSpec((_BE, _H), row)],
        out_shape=[jax.ShapeDtypeStruct((_E, _NH), jnp.float32),
                   jax.ShapeDtypeStruct((_E, _H), jnp.float32)],
    )(e, zd, vs, expand)


def _gate_kernel(x_ref, m_ref, w2_ref, b2_ref, out_ref):
    g1 = jnp.maximum(x_ref[...], 0.0)
    g = jax.nn.sigmoid(_dot(g1, w2_ref[...]) + b2_ref[...])
    out_ref[...] = g * m_ref[...]


def _gate_mm(x, m, W2, b2, block):
    rows = x.shape[0]
    grid = rows // block
    row = lambda i: (i, 0)
    full = lambda i: (0, 0)
    return pl.pallas_call(
        _gate_kernel,
        grid=(grid,),
        in_specs=[pl.BlockSpec((block, _H), row), pl.BlockSpec((block, _H), row),
                  pl.BlockSpec((_H, _H), full), pl.BlockSpec((1, _H), full)],
        out_specs=pl.BlockSpec((block, _H), row),
        out_shape=jax.ShapeDtypeStruct((rows, _H), jnp.float32),
    )(x, m, W2, b2.reshape(1, _H))


def _combine_kernel(h_ref, ag_ref, up_ref, wout_ref, wc1_ref, wc2_ref,
                    bc_ref, out_ref):
    ds = _dot(ag_ref[...], wout_ref[...])
    out_ref[...] = (h_ref[...] + _dot(ds, wc1_ref[...])
                    + _dot(up_ref[...], wc2_ref[...]) + bc_ref[...])


def _combine(h, aggr, upstream, W_out, W_comb, b_comb):
    grid = _N // _BN
    row = lambda i: (i, 0)
    full = lambda i: (0, 0)
    return pl.pallas_call(
        _combine_kernel,
        grid=(grid,),
        in_specs=[pl.BlockSpec((_BN, _H), row)] * 3
        + [pl.BlockSpec((_H, _H), full)] * 3
        + [pl.BlockSpec((1, _H), full)],
        out_specs=pl.BlockSpec((_BN, _H), row),
        out_shape=jax.ShapeDtypeStruct((_N, _H), jnp.float32),
    )(h, aggr, upstream, W_out, W_comb[:_H], W_comb[_H:],
      b_comb.reshape(1, _H))


def kernel(h, x_s, edge_index, node_mask, edge_mask, W_q, W_k, W_v, W_static, W_out, ln_ds_w, ln_ds_b, msg_W1, msg_b1, msg_W2, msg_b2, gate_W1, gate_b1, gate_W2, gate_b2, ln_w, ln_b, W_comb, b_comb):
    src = edge_index[0]
    dest = edge_index[1]
    q, k, v, pm, ps, pr = _node_precompute(
        h, x_s, W_q, W_k, W_v, W_static, ln_ds_w, ln_ds_b,
        msg_W1, msg_b1, msg_W2, msg_b2, gate_W1, gate_b1, ln_w, ln_b)

    qd = q[dest]
    ks = k[src]
    vs = v[src]
    e = _escore(qd, ks)
    z = jax.ops.segment_sum(e, dest, num_segments=_N)
    zd = z[dest]
    attn, wv = _attn_wv(e, zd, vs)
    aggr = jax.ops.segment_sum(wv, dest, num_segments=_N)

    gpre = ps[dest] + pr[src]
    pmd = pm[dest]
    gated = _gate_mm(gpre, pmd, gate_W2, gate_b2, _BE)
    upstream = jax.ops.segment_sum(gated, src, num_segments=_N)

    winner = jnp.full((_N,), -1, jnp.int32).at[src].max(
        jnp.arange(_E, dtype=jnp.int32))
    has = winner >= 0
    partner = dest[jnp.where(has, winner, 0)]
    gpn_in = ps[partner] + pr
    hasf = jnp.broadcast_to(has[:, None], (_N, _H)).astype(jnp.float32)
    gpn = _gate_mm(gpn_in, hasf, gate_W2, gate_b2, _BN)

    out = _combine(h, aggr, upstream, W_out, W_comb, b_comb)
    return (out, attn, gpn)


# speedup vs baseline: 2.0569x; 1.1660x over previous
import functools

import jax
import jax.numpy as jnp
import numpy as np
from jax import lax
from jax.experimental import pallas as pl
from jax.experimental.pallas import tpu as pltpu
from jax.experimental.pallas import tpu_sc as plsc

_N = 10000
_E = 160000
_H = 256
_NH = 8
_HD = 32
_S = 16

_BN = 1000
_BE = 2000


def _lnorm(x, w, b):
    m = jnp.mean(x, axis=-1, keepdims=True)
    v = jnp.mean((x - m) ** 2, axis=-1, keepdims=True)
    return (x - m) / jnp.sqrt(v + 1e-5) * w + b


def _dot(a, b):
    return jnp.dot(a, b, preferred_element_type=jnp.float32)


def _node_kernel(h_ref, xs_ref, wq_ref, wk_ref, wv_ref, wstat_ref,
                 lnds_w_ref, lnds_b_ref, m1h_ref, m1x_ref, mb1_ref,
                 m2_ref, mb2_ref, gs_ref, gr_ref, gxs_ref, gxr_ref,
                 gb1_ref, lnw_ref, lnb_ref,
                 q_ref, k_ref, v_ref, pm_ref, ps_ref, pr_ref):
    h = h_ref[...]
    xs = xs_ref[...]
    hn = _lnorm(h, lnw_ref[...], lnb_ref[...])
    hc = _lnorm(hn + _dot(xs, wstat_ref[...]), lnds_w_ref[...], lnds_b_ref[...])
    q_ref[...] = _dot(hc, wq_ref[...])
    k_ref[...] = _dot(hc, wk_ref[...])
    v_ref[...] = _dot(hc, wv_ref[...])
    m1 = jnp.maximum(_dot(hn, m1h_ref[...]) + _dot(xs, m1x_ref[...]) + mb1_ref[...], 0.0)
    pm_ref[...] = _dot(m1, m2_ref[...]) + mb2_ref[...]
    ps_ref[...] = _dot(hn, gs_ref[...]) + _dot(xs, gxs_ref[...]) + gb1_ref[...]
    pr_ref[...] = _dot(hn, gr_ref[...]) + _dot(xs, gxr_ref[...])


def _node_precompute(h, x_s, W_q, W_k, W_v, W_static, ln_ds_w, ln_ds_b,
                     msg_W1, msg_b1, msg_W2, msg_b2, gate_W1, gate_b1,
                     ln_w, ln_b):
    grid = _N // _BN
    row = lambda i: (i, 0)
    full = lambda i: (0, 0)
    hspec = pl.BlockSpec((_BN, _H), row)
    xspec = pl.BlockSpec((_BN, _S), row)
    wspec = pl.BlockSpec((_H, _H), full)
    sspec = pl.BlockSpec((_S, _H), full)
    bspec = pl.BlockSpec((1, _H), full)
    out = [jax.ShapeDtypeStruct((_N, _H), jnp.float32) for _ in range(6)]
    return pl.pallas_call(
        _node_kernel,
        grid=(grid,),
        in_specs=[hspec, xspec, wspec, wspec, wspec, sspec, bspec, bspec,
                  wspec, sspec, bspec, wspec, bspec, wspec, wspec, sspec,
                  sspec, bspec, bspec, bspec],
        out_specs=[hspec] * 6,
        out_shape=out,
    )(h, x_s, W_q, W_k, W_v, W_static,
      ln_ds_w.reshape(1, _H), ln_ds_b.reshape(1, _H),
      msg_W1[:_H], msg_W1[_H:], msg_b1.reshape(1, _H), msg_W2,
      msg_b2.reshape(1, _H),
      gate_W1[:_H], gate_W1[_H:2 * _H], gate_W1[2 * _H:2 * _H + _S],
      gate_W1[2 * _H + _S:], gate_b1.reshape(1, _H),
      ln_w.reshape(1, _H), ln_b.reshape(1, _H))


def _escore_kernel(qd_ref, ks_ref, sel_ref, e_ref):
    prod = qd_ref[...] * ks_ref[...]
    s = _dot(prod, sel_ref[...]) * (1.0 / np.sqrt(float(_HD)))
    e_ref[...] = jnp.exp(s)


def _escore(qd, ks):
    sel = jnp.repeat(jnp.eye(_NH, dtype=jnp.float32), _HD, axis=0)
    grid = _E // _BE
    row = lambda i: (i, 0)
    return pl.pallas_call(
        _escore_kernel,
        grid=(grid,),
        in_specs=[pl.BlockSpec((_BE, _H), row), pl.BlockSpec((_BE, _H), row),
                  pl.BlockSpec((_H, _NH), lambda i: (0, 0))],
        out_specs=pl.BlockSpec((_BE, _NH), row),
        out_shape=jax.ShapeDtypeStruct((_E, _NH), jnp.float32),
    )(qd, ks, sel)


def _attn_kernel(e_ref, zd_ref, vs_ref, exp_ref, attn_ref, wv_ref):
    attn = e_ref[...] / (zd_ref[...] + 1e-9)
    attn_ref[...] = attn
    wv_ref[...] = vs_ref[...] * _dot(attn, exp_ref[...])


def _attn_wv(e, zd, vs):
    expand = jnp.repeat(jnp.eye(_NH, dtype=jnp.float32), _HD, axis=1)
    grid = _E // _BE
    row = lambda i: (i, 0)
    return pl.pallas_call(
        _attn_kernel,
        grid=(grid,),
        in_specs=[pl.BlockSpec((_BE, _NH), row), pl.BlockSpec((_BE, _NH), row),
                  pl.BlockSpec((_BE, _H), row),
                  pl.BlockSpec((_NH, _H), lambda i: (0, 0))],
        out_specs=[pl.BlockSpec((_BE, _NH), row), pl.BlockSpec((_BE, _H), row)],
        out_shape=[jax.ShapeDtypeStruct((_E, _NH), jnp.float32),
                   jax.ShapeDtypeStruct((_E, _H), jnp.float32)],
    )(e, zd, vs, expand)


def _gate_kernel(x_ref, m_ref, w2_ref, b2_ref, out_ref):
    g1 = jnp.maximum(x_ref[...], 0.0)
    g = jax.nn.sigmoid(_dot(g1, w2_ref[...]) + b2_ref[...])
    out_ref[...] = g * m_ref[...]


def _gate_mm(x, m, W2, b2, block):
    rows = x.shape[0]
    grid = rows // block
    row = lambda i: (i, 0)
    full = lambda i: (0, 0)
    return pl.pallas_call(
        _gate_kernel,
        grid=(grid,),
        in_specs=[pl.BlockSpec((block, _H), row), pl.BlockSpec((block, _H), row),
                  pl.BlockSpec((_H, _H), full), pl.BlockSpec((1, _H), full)],
        out_specs=pl.BlockSpec((block, _H), row),
        out_shape=jax.ShapeDtypeStruct((rows, _H), jnp.float32),
    )(x, m, W2, b2.reshape(1, _H))


_SC_BLK = 80
_SC_EPT = _E // 16
_SC_NBLK = _SC_EPT // _SC_BLK
_SC_ACC = _N + 8
_SC_ZPT = 624


def _sc_segsum_body(vals_hbm, idx_hbm, zeros_hbm, out_hbm,
                    idx0, idx1, valb0, valb1, acc, sems):
    c = lax.axis_index("c")
    s = lax.axis_index("s")
    col = c * 128
    ebase = s * _SC_EPT
    idxb = (idx0, idx1)
    valb = (valb0, valb1)

    @pl.when(s < 15)
    def _():
        pltpu.sync_copy(zeros_hbm.at[pl.ds(0, _SC_ZPT)],
                        acc.at[pl.ds(s * _SC_ZPT, _SC_ZPT)])

    @pl.when(s == 15)
    def _():
        pltpu.sync_copy(zeros_hbm,
                        acc.at[pl.ds(15 * _SC_ZPT, _SC_ACC - 15 * _SC_ZPT)])

    plsc.subcore_barrier()

    def _copies(g, slot):
        return [
            pltpu.make_async_copy(
                vals_hbm.at[pl.ds(ebase + g * _SC_BLK, _SC_BLK),
                            pl.ds(col, 128)],
                valb[slot], sems.at[slot]),
            pltpu.make_async_copy(
                idx_hbm.at[pl.ds(ebase + g * _SC_BLK, _SC_BLK)],
                idxb[slot], sems.at[slot]),
        ]

    def _start(g, slot):
        for cp in _copies(g, slot):
            cp.start()

    def _wait(g, slot):
        for cp in _copies(g, slot):
            cp.wait()

    def _scatter(slot):
        pltpu.sync_copy(valb[slot], acc.at[idxb[slot]], add=True)

    _start(0, 0)

    def _body(i, carry):
        g0 = 2 * i
        _wait(g0, 0)
        _start(g0 + 1, 1)
        _scatter(0)
        _wait(g0 + 1, 1)
        _start(g0 + 2, 0)
        _scatter(1)
        return carry

    lax.fori_loop(0, (_SC_NBLK - 1) // 2, _body, 0)
    _wait(_SC_NBLK - 1, 0)
    _scatter(0)

    plsc.subcore_barrier()

    @pl.when(s < 15)
    def _():
        pltpu.sync_copy(
            acc.at[pl.ds(s * _SC_ZPT, _SC_ZPT)],
            out_hbm.at[pl.ds(s * _SC_ZPT, _SC_ZPT), pl.ds(col, 128)])

    @pl.when(s == 15)
    def _():
        pltpu.sync_copy(
            acc.at[pl.ds(15 * _SC_ZPT, _N - 15 * _SC_ZPT)],
            out_hbm.at[pl.ds(15 * _SC_ZPT, _N - 15 * _SC_ZPT),
                       pl.ds(col, 128)])


def _sc_segsum(vals, idx):
    zeros = jnp.zeros((_SC_ACC - 15 * _SC_ZPT, 128), jnp.float32)
    mesh = plsc.VectorSubcoreMesh(core_axis_name="c", subcore_axis_name="s")
    f = pl.kernel(
        _sc_segsum_body,
        out_type=jax.ShapeDtypeStruct((_N, _H), jnp.float32),
        mesh=mesh,
        scratch_types=[
            pltpu.VMEM((_SC_BLK,), jnp.int32),
            pltpu.VMEM((_SC_BLK,), jnp.int32),
            pltpu.VMEM((_SC_BLK, 128), jnp.float32),
            pltpu.VMEM((_SC_BLK, 128), jnp.float32),
            pltpu.VMEM_SHARED((_SC_ACC, 128), jnp.float32),
            pltpu.SemaphoreType.DMA((2,)),
        ],
    )
    return f(vals, idx.astype(jnp.int32), zeros)


def _combine_kernel(h_ref, ag_ref, up_ref, wout_ref, wc1_ref, wc2_ref,
                    bc_ref, out_ref):
    ds = _dot(ag_ref[...], wout_ref[...])
    out_ref[...] = (h_ref[...] + _dot(ds, wc1_ref[...])
                    + _dot(up_ref[...], wc2_ref[...]) + bc_ref[...])


def _combine(h, aggr, upstream, W_out, W_comb, b_comb):
    grid = _N // _BN
    row = lambda i: (i, 0)
    full = lambda i: (0, 0)
    return pl.pallas_call(
        _combine_kernel,
        grid=(grid,),
        in_specs=[pl.BlockSpec((_BN, _H), row)] * 3
        + [pl.BlockSpec((_H, _H), full)] * 3
        + [pl.BlockSpec((1, _H), full)],
        out_specs=pl.BlockSpec((_BN, _H), row),
        out_shape=jax.ShapeDtypeStruct((_N, _H), jnp.float32),
    )(h, aggr, upstream, W_out, W_comb[:_H], W_comb[_H:],
      b_comb.reshape(1, _H))


def kernel(h, x_s, edge_index, node_mask, edge_mask, W_q, W_k, W_v, W_static, W_out, ln_ds_w, ln_ds_b, msg_W1, msg_b1, msg_W2, msg_b2, gate_W1, gate_b1, gate_W2, gate_b2, ln_w, ln_b, W_comb, b_comb):
    src = edge_index[0]
    dest = edge_index[1]
    q, k, v, pm, ps, pr = _node_precompute(
        h, x_s, W_q, W_k, W_v, W_static, ln_ds_w, ln_ds_b,
        msg_W1, msg_b1, msg_W2, msg_b2, gate_W1, gate_b1, ln_w, ln_b)

    qd = q[dest]
    ks = k[src]
    vs = v[src]
    e = _escore(qd, ks)
    z = jax.ops.segment_sum(e, dest, num_segments=_N)
    zd = z[dest]
    attn, wv = _attn_wv(e, zd, vs)
    aggr = _sc_segsum(wv, dest)

    gpre = ps[dest] + pr[src]
    pmd = pm[dest]
    gated = _gate_mm(gpre, pmd, gate_W2, gate_b2, _BE)
    upstream = _sc_segsum(gated, src)

    winner = jnp.full((_N,), -1, jnp.int32).at[src].max(
        jnp.arange(_E, dtype=jnp.int32))
    has = winner >= 0
    partner = dest[jnp.where(has, winner, 0)]
    gpn_in = ps[partner] + pr
    hasf = jnp.broadcast_to(has[:, None], (_N, _H)).astype(jnp.float32)
    gpn = _gate_mm(gpn_in, hasf, gate_W2, gate_b2, _BN)

    out = _combine(h, aggr, upstream, W_out, W_comb, b_comb)
    return (out, attn, gpn)


# speedup vs baseline: 2.1262x; 1.0337x over previous
import functools

import jax
import jax.numpy as jnp
import numpy as np
from jax import lax
from jax.experimental import pallas as pl
from jax.experimental.pallas import tpu as pltpu
from jax.experimental.pallas import tpu_sc as plsc

_N = 10000
_E = 160000
_H = 256
_NH = 8
_HD = 32
_S = 16

_BN = 1000
_BE = 2000


def _lnorm(x, w, b):
    m = jnp.mean(x, axis=-1, keepdims=True)
    v = jnp.mean((x - m) ** 2, axis=-1, keepdims=True)
    return (x - m) / jnp.sqrt(v + 1e-5) * w + b


def _dot(a, b):
    return jnp.dot(a, b, preferred_element_type=jnp.float32)


def _node_kernel(h_ref, xs_ref, wq_ref, wk_ref, wv_ref, wstat_ref,
                 lnds_w_ref, lnds_b_ref, m1h_ref, m1x_ref, mb1_ref,
                 m2_ref, mb2_ref, gs_ref, gr_ref, gxs_ref, gxr_ref,
                 gb1_ref, lnw_ref, lnb_ref,
                 q_ref, k_ref, v_ref, pm_ref, ps_ref, pr_ref):
    h = h_ref[...]
    xs = xs_ref[...]
    hn = _lnorm(h, lnw_ref[...], lnb_ref[...])
    hc = _lnorm(hn + _dot(xs, wstat_ref[...]), lnds_w_ref[...], lnds_b_ref[...])
    q_ref[...] = _dot(hc, wq_ref[...])
    k_ref[...] = _dot(hc, wk_ref[...])
    v_ref[...] = _dot(hc, wv_ref[...])
    m1 = jnp.maximum(_dot(hn, m1h_ref[...]) + _dot(xs, m1x_ref[...]) + mb1_ref[...], 0.0)
    pm_ref[...] = _dot(m1, m2_ref[...]) + mb2_ref[...]
    ps_ref[...] = _dot(hn, gs_ref[...]) + _dot(xs, gxs_ref[...]) + gb1_ref[...]
    pr_ref[...] = _dot(hn, gr_ref[...]) + _dot(xs, gxr_ref[...])


def _node_precompute(h, x_s, W_q, W_k, W_v, W_static, ln_ds_w, ln_ds_b,
                     msg_W1, msg_b1, msg_W2, msg_b2, gate_W1, gate_b1,
                     ln_w, ln_b):
    grid = _N // _BN
    row = lambda i: (i, 0)
    full = lambda i: (0, 0)
    hspec = pl.BlockSpec((_BN, _H), row)
    xspec = pl.BlockSpec((_BN, _S), row)
    wspec = pl.BlockSpec((_H, _H), full)
    sspec = pl.BlockSpec((_S, _H), full)
    bspec = pl.BlockSpec((1, _H), full)
    out = [jax.ShapeDtypeStruct((_N, _H), jnp.float32) for _ in range(6)]
    return pl.pallas_call(
        _node_kernel,
        grid=(grid,),
        in_specs=[hspec, xspec, wspec, wspec, wspec, sspec, bspec, bspec,
                  wspec, sspec, bspec, wspec, bspec, wspec, wspec, sspec,
                  sspec, bspec, bspec, bspec],
        out_specs=[hspec] * 6,
        out_shape=out,
    )(h, x_s, W_q, W_k, W_v, W_static,
      ln_ds_w.reshape(1, _H), ln_ds_b.reshape(1, _H),
      msg_W1[:_H], msg_W1[_H:], msg_b1.reshape(1, _H), msg_W2,
      msg_b2.reshape(1, _H),
      gate_W1[:_H], gate_W1[_H:2 * _H], gate_W1[2 * _H:2 * _H + _S],
      gate_W1[2 * _H + _S:], gate_b1.reshape(1, _H),
      ln_w.reshape(1, _H), ln_b.reshape(1, _H))


_EC = 16


def _escore_kernel(qd_ref, ks_ref, sel_ref, e16_ref, e128_ref):
    prod = qd_ref[...] * ks_ref[...]
    s = _dot(prod, sel_ref[...]) * (1.0 / np.sqrt(float(_HD)))
    ex = jnp.exp(s)
    e16_ref[...] = ex[:, :_EC]
    e128_ref[...] = ex


def _escore(qd, ks):
    sel = jnp.repeat(jnp.eye(_NH, dtype=jnp.float32), _HD, axis=0)
    sel = jnp.concatenate(
        [sel, jnp.zeros((_H, 128 - _NH), jnp.float32)], axis=1)
    grid = _E // _BE
    row = lambda i: (i, 0)
    return pl.pallas_call(
        _escore_kernel,
        grid=(grid,),
        in_specs=[pl.BlockSpec((_BE, _H), row), pl.BlockSpec((_BE, _H), row),
                  pl.BlockSpec((_H, 128), lambda i: (0, 0))],
        out_specs=[pl.BlockSpec((_BE, _EC), row),
                   pl.BlockSpec((_BE, 128), row)],
        out_shape=[jax.ShapeDtypeStruct((_E, _EC), jnp.float32),
                   jax.ShapeDtypeStruct((_E, 128), jnp.float32)],
    )(qd, ks, sel)


def _wvun_kernel(e_ref, vs_ref, exp_ref, wv_ref):
    wv_ref[...] = vs_ref[...] * _dot(e_ref[:, :_NH], exp_ref[...])


def _wv_un(e16, vs):
    expand = jnp.repeat(jnp.eye(_NH, dtype=jnp.float32), _HD, axis=1)
    grid = _E // _BE
    row = lambda i: (i, 0)
    return pl.pallas_call(
        _wvun_kernel,
        grid=(grid,),
        in_specs=[pl.BlockSpec((_BE, _EC), row), pl.BlockSpec((_BE, _H), row),
                  pl.BlockSpec((_NH, _H), lambda i: (0, 0))],
        out_specs=pl.BlockSpec((_BE, _H), row),
        out_shape=jax.ShapeDtypeStruct((_E, _H), jnp.float32),
    )(e16, vs, expand)


def _attn_kernel(e_ref, zd_ref, attn_ref):
    attn_ref[...] = e_ref[:, :_NH] / (zd_ref[:, :_NH] + 1e-9)


def _attn(e16, zd16):
    grid = _E // _BE
    row = lambda i: (i, 0)
    return pl.pallas_call(
        _attn_kernel,
        grid=(grid,),
        in_specs=[pl.BlockSpec((_BE, _EC), row), pl.BlockSpec((_BE, _EC), row)],
        out_specs=pl.BlockSpec((_BE, _NH), row),
        out_shape=jax.ShapeDtypeStruct((_E, _NH), jnp.float32),
    )(e16, zd16)


def _gate_kernel(x_ref, m_ref, w2_ref, b2_ref, out_ref):
    g1 = jnp.maximum(x_ref[...], 0.0)
    g = jax.nn.sigmoid(_dot(g1, w2_ref[...]) + b2_ref[...])
    out_ref[...] = g * m_ref[...]


def _gate_mm(x, m, W2, b2, block):
    rows = x.shape[0]
    grid = rows // block
    row = lambda i: (i, 0)
    full = lambda i: (0, 0)
    return pl.pallas_call(
        _gate_kernel,
        grid=(grid,),
        in_specs=[pl.BlockSpec((block, _H), row), pl.BlockSpec((block, _H), row),
                  pl.BlockSpec((_H, _H), full), pl.BlockSpec((1, _H), full)],
        out_specs=pl.BlockSpec((block, _H), row),
        out_shape=jax.ShapeDtypeStruct((rows, _H), jnp.float32),
    )(x, m, W2, b2.reshape(1, _H))


_SC_BLK = 80
_SC_EPT = _E // 16
_SC_NBLK = _SC_EPT // _SC_BLK
_SC_ACC = _N + 8
_SC_ZPT = 624


def _sc_segsum_body(vals_hbm, idx_hbm, zeros_hbm, out_hbm,
                    idx0, idx1, valb0, valb1, acc, sems):
    c = lax.axis_index("c")
    s = lax.axis_index("s")
    col = c * 128
    ebase = s * _SC_EPT
    idxb = (idx0, idx1)
    valb = (valb0, valb1)

    @pl.when(s < 15)
    def _():
        pltpu.sync_copy(zeros_hbm.at[pl.ds(0, _SC_ZPT)],
                        acc.at[pl.ds(s * _SC_ZPT, _SC_ZPT)])

    @pl.when(s == 15)
    def _():
        pltpu.sync_copy(zeros_hbm,
                        acc.at[pl.ds(15 * _SC_ZPT, _SC_ACC - 15 * _SC_ZPT)])

    plsc.subcore_barrier()

    def _copies(g, slot):
        return [
            pltpu.make_async_copy(
                vals_hbm.at[pl.ds(ebase + g * _SC_BLK, _SC_BLK),
                            pl.ds(col, 128)],
                valb[slot], sems.at[slot]),
            pltpu.make_async_copy(
                idx_hbm.at[pl.ds(ebase + g * _SC_BLK, _SC_BLK)],
                idxb[slot], sems.at[slot]),
        ]

    def _start(g, slot):
        for cp in _copies(g, slot):
            cp.start()

    def _wait(g, slot):
        for cp in _copies(g, slot):
            cp.wait()

    def _scatter(slot):
        pltpu.sync_copy(valb[slot], acc.at[idxb[slot]], add=True)

    _start(0, 0)

    def _body(i, carry):
        g0 = 2 * i
        _wait(g0, 0)
        _start(g0 + 1, 1)
        _scatter(0)
        _wait(g0 + 1, 1)
        _start(g0 + 2, 0)
        _scatter(1)
        return carry

    lax.fori_loop(0, (_SC_NBLK - 1) // 2, _body, 0)
    _wait(_SC_NBLK - 1, 0)
    _scatter(0)

    plsc.subcore_barrier()

    @pl.when(s < 15)
    def _():
        pltpu.sync_copy(
            acc.at[pl.ds(s * _SC_ZPT, _SC_ZPT)],
            out_hbm.at[pl.ds(s * _SC_ZPT, _SC_ZPT), pl.ds(col, 128)])

    @pl.when(s == 15)
    def _():
        pltpu.sync_copy(
            acc.at[pl.ds(15 * _SC_ZPT, _N - 15 * _SC_ZPT)],
            out_hbm.at[pl.ds(15 * _SC_ZPT, _N - 15 * _SC_ZPT),
                       pl.ds(col, 128)])


_SCZ_BLK = 40
_SCZ_EPT = _E // 32
_SCZ_NBLK = _SCZ_EPT // _SCZ_BLK


def _sc_zsum_body(e_hbm, idx_hbm, zeros_hbm, out_hbm,
                  idx0, idx1, eb0, eb1, acc, sems):
    c = lax.axis_index("c")
    s = lax.axis_index("s")
    col = c * 128
    ebase = (c * 16 + s) * _SCZ_EPT
    idxb = (idx0, idx1)
    eb = (eb0, eb1)

    @pl.when(s < 15)
    def _():
        pltpu.sync_copy(zeros_hbm.at[pl.ds(0, _SC_ZPT)],
                        acc.at[pl.ds(s * _SC_ZPT, _SC_ZPT)])

    @pl.when(s == 15)
    def _():
        pltpu.sync_copy(zeros_hbm,
                        acc.at[pl.ds(15 * _SC_ZPT, _SC_ACC - 15 * _SC_ZPT)])

    plsc.subcore_barrier()

    def _copies(g, slot):
        return [
            pltpu.make_async_copy(
                e_hbm.at[pl.ds(ebase + g * _SCZ_BLK, _SCZ_BLK)],
                eb[slot], sems.at[slot]),
            pltpu.make_async_copy(
                idx_hbm.at[pl.ds(ebase + g * _SCZ_BLK, _SCZ_BLK)],
                idxb[slot], sems.at[slot]),
        ]

    def _start(g, slot):
        for cp in _copies(g, slot):
            cp.start()

    def _wait(g, slot):
        for cp in _copies(g, slot):
            cp.wait()

    def _scatter(slot):
        pltpu.sync_copy(eb[slot], acc.at[idxb[slot]], add=True)

    _start(0, 0)

    def _body(i, carry):
        g0 = 2 * i
        _wait(g0, 0)
        _start(g0 + 1, 1)
        _scatter(0)
        _wait(g0 + 1, 1)
        _start(g0 + 2, 0)
        _scatter(1)
        return carry

    lax.fori_loop(0, (_SCZ_NBLK - 1) // 2, _body, 0)
    _wait(_SCZ_NBLK - 1, 0)
    _scatter(0)

    plsc.subcore_barrier()

    @pl.when(s < 15)
    def _():
        pltpu.sync_copy(
            acc.at[pl.ds(s * _SC_ZPT, _SC_ZPT)],
            out_hbm.at[pl.ds(s * _SC_ZPT, _SC_ZPT), pl.ds(col, 128)])

    @pl.when(s == 15)
    def _():
        pltpu.sync_copy(
            acc.at[pl.ds(15 * _SC_ZPT, _N - 15 * _SC_ZPT)],
            out_hbm.at[pl.ds(15 * _SC_ZPT, _N - 15 * _SC_ZPT),
                       pl.ds(col, 128)])


def _sc_zsum(e128, idx):
    zeros = jnp.zeros((_SC_ACC - 15 * _SC_ZPT, 128), jnp.float32)
    mesh = plsc.VectorSubcoreMesh(core_axis_name="c", subcore_axis_name="s")
    f = pl.kernel(
        _sc_zsum_body,
        out_type=jax.ShapeDtypeStruct((_N, 256), jnp.float32),
        mesh=mesh,
        scratch_types=[
            pltpu.VMEM((_SCZ_BLK,), jnp.int32),
            pltpu.VMEM((_SCZ_BLK,), jnp.int32),
            pltpu.VMEM((_SCZ_BLK, 128), jnp.float32),
            pltpu.VMEM((_SCZ_BLK, 128), jnp.float32),
            pltpu.VMEM_SHARED((_SC_ACC, 128), jnp.float32),
            pltpu.SemaphoreType.DMA((2,)),
        ],
    )
    return f(e128, idx.astype(jnp.int32), zeros)


def _sc_segsum(vals, idx):
    zeros = jnp.zeros((_SC_ACC - 15 * _SC_ZPT, 128), jnp.float32)
    mesh = plsc.VectorSubcoreMesh(core_axis_name="c", subcore_axis_name="s")
    f = pl.kernel(
        _sc_segsum_body,
        out_type=jax.ShapeDtypeStruct((_N, _H), jnp.float32),
        mesh=mesh,
        scratch_types=[
            pltpu.VMEM((_SC_BLK,), jnp.int32),
            pltpu.VMEM((_SC_BLK,), jnp.int32),
            pltpu.VMEM((_SC_BLK, 128), jnp.float32),
            pltpu.VMEM((_SC_BLK, 128), jnp.float32),
            pltpu.VMEM_SHARED((_SC_ACC, 128), jnp.float32),
            pltpu.SemaphoreType.DMA((2,)),
        ],
    )
    return f(vals, idx.astype(jnp.int32), zeros)


_SCW_EPT = _E // 32
_SCW_VREGS = _SCW_EPT // 16
_SCW_TAIL = _SCW_EPT - 16 * _SCW_VREGS


def _sc_winner_body(idx_hbm, out_hbm, ibuf, tbl, sem):
    c = lax.axis_index("c")
    s = lax.axis_index("s")
    w = c * 16 + s
    base = w * _SCW_EPT

    ibuf[pl.ds(_SCW_EPT - _SCW_TAIL, 16)] = jnp.zeros((16,), jnp.int32)
    cp = pltpu.make_async_copy(idx_hbm.at[pl.ds(base, _SCW_EPT)],
                               ibuf.at[pl.ds(0, _SCW_EPT)], sem)
    cp.start()

    def _init(i, carry):
        tbl[pl.ds(i * 16, 16)] = jnp.full((16,), -1, jnp.int32)
        return carry

    lax.fori_loop(0, _N // 16, _init, 0)
    cp.wait()

    lanes = lax.iota(jnp.int32, 16)
    masks = [lanes == l for l in range(16)]

    def _step(j, carry):
        iv = ibuf[pl.ds(j * 16, 16)]
        ids = lanes + (base + j * 16)
        for l in range(16):
            plsc.store_scatter(tbl, (iv,), ids, mask=masks[l])
        return carry

    lax.fori_loop(0, _SCW_VREGS, _step, 0)
    ivt = ibuf[pl.ds(_SCW_VREGS * 16, 16)]
    idt = lanes + (base + _SCW_VREGS * 16)
    valid = lanes < _SCW_TAIL
    for l in range(_SCW_TAIL):
        plsc.store_scatter(tbl, (ivt,), idt, mask=masks[l] & valid)

    pltpu.sync_copy(tbl, out_hbm.at[pl.ds(w * _N, _N)])


def _sc_winner(idx):
    mesh = plsc.VectorSubcoreMesh(core_axis_name="c", subcore_axis_name="s")
    f = pl.kernel(
        _sc_winner_body,
        out_type=jax.ShapeDtypeStruct((32 * _N,), jnp.int32),
        mesh=mesh,
        scratch_types=[
            pltpu.VMEM((_SCW_EPT + 16,), jnp.int32),
            pltpu.VMEM((_N,), jnp.int32),
            pltpu.SemaphoreType.DMA,
        ],
    )
    return f(idx.astype(jnp.int32))


def _combine_kernel(h_ref, ag_ref, z_ref, up_ref, exp_ref, wout_ref,
                    wc1_ref, wc2_ref, bc_ref, out_ref):
    recip = 1.0 / (z_ref[:, :_NH] + 1e-9)
    ag = ag_ref[...] * _dot(recip, exp_ref[...])
    ds = _dot(ag, wout_ref[...])
    out_ref[...] = (h_ref[...] + _dot(ds, wc1_ref[...])
                    + _dot(up_ref[...], wc2_ref[...]) + bc_ref[...])


def _combine(h, aggr_un, z16, upstream, W_out, W_comb, b_comb):
    expand = jnp.repeat(jnp.eye(_NH, dtype=jnp.float32), _HD, axis=1)
    grid = _N // _BN
    row = lambda i: (i, 0)
    full = lambda i: (0, 0)
    return pl.pallas_call(
        _combine_kernel,
        grid=(grid,),
        in_specs=[pl.BlockSpec((_BN, _H), row), pl.BlockSpec((_BN, _H), row),
                  pl.BlockSpec((_BN, _EC), row), pl.BlockSpec((_BN, _H), row),
                  pl.BlockSpec((_NH, _H), full)]
        + [pl.BlockSpec((_H, _H), full)] * 3
        + [pl.BlockSpec((1, _H), full)],
        out_specs=pl.BlockSpec((_BN, _H), row),
        out_shape=jax.ShapeDtypeStruct((_N, _H), jnp.float32),
    )(h, aggr_un, z16, upstream, expand, W_out, W_comb[:_H], W_comb[_H:],
      b_comb.reshape(1, _H))


def kernel(h, x_s, edge_index, node_mask, edge_mask, W_q, W_k, W_v, W_static, W_out, ln_ds_w, ln_ds_b, msg_W1, msg_b1, msg_W2, msg_b2, gate_W1, gate_b1, gate_W2, gate_b2, ln_w, ln_b, W_comb, b_comb):
    src = edge_index[0]
    dest = edge_index[1]
    q, k, v, pm, ps, pr = _node_precompute(
        h, x_s, W_q, W_k, W_v, W_static, ln_ds_w, ln_ds_b,
        msg_W1, msg_b1, msg_W2, msg_b2, gate_W1, gate_b1, ln_w, ln_b)

    qd = q[dest]
    ks = k[src]
    vs = v[src]
    e16, e128 = _escore(qd, ks)
    wv = _wv_un(e16, vs)
    aggr_un = _sc_segsum(wv, dest)
    zpart = _sc_zsum(e128, dest)
    z16 = zpart[:, :_EC] + zpart[:, 128:128 + _EC]
    zd = z16[dest]
    attn = _attn(e16, zd)

    gpre = ps[dest] + pr[src]
    pmd = pm[dest]
    gated = _gate_mm(gpre, pmd, gate_W2, gate_b2, _BE)
    upstream = _sc_segsum(gated, src)

    winner = jnp.full((_N,), -1, jnp.int32).at[src].max(
        jnp.arange(_E, dtype=jnp.int32))
    has = winner >= 0
    partner = dest[jnp.where(has, winner, 0)]
    gpn_in = ps[partner] + pr
    hasf = jnp.broadcast_to(has[:, None], (_N, _H)).astype(jnp.float32)
    gpn = _gate_mm(gpn_in, hasf, gate_W2, gate_b2, _BN)

    out = _combine(h, aggr_un, z16, upstream, W_out, W_comb, b_comb)
    return (out, attn, gpn)


# speedup vs baseline: 3.1068x; 1.4612x over previous
import functools

import jax
import jax.numpy as jnp
import numpy as np
from jax import lax
from jax.experimental import pallas as pl
from jax.experimental.pallas import tpu as pltpu
from jax.experimental.pallas import tpu_sc as plsc

_N = 10000
_E = 160000
_H = 256
_NH = 8
_HD = 32
_S = 16

_BN = 1000
_BE = 2000


def _lnorm(x, w, b):
    m = jnp.mean(x, axis=-1, keepdims=True)
    v = jnp.mean((x - m) ** 2, axis=-1, keepdims=True)
    return (x - m) / jnp.sqrt(v + 1e-5) * w + b


def _dot(a, b):
    return jnp.dot(a, b, preferred_element_type=jnp.float32)


def _node_kernel(h_ref, xs_ref, wq_ref, wk_ref, wv_ref, wstat_ref,
                 lnds_w_ref, lnds_b_ref, m1h_ref, m1x_ref, mb1_ref,
                 m2_ref, mb2_ref, gs_ref, gr_ref, gxs_ref, gxr_ref,
                 gb1_ref, lnw_ref, lnb_ref,
                 td_ref, ts_ref, ps_ref):
    h = h_ref[...]
    xs = xs_ref[...]
    hn = _lnorm(h, lnw_ref[...], lnb_ref[...])
    hc = _lnorm(hn + _dot(xs, wstat_ref[...]), lnds_w_ref[...], lnds_b_ref[...])
    ps = _dot(hn, gs_ref[...]) + _dot(xs, gxs_ref[...]) + gb1_ref[...]
    td_ref[:, 0:_H] = ps
    ps_ref[...] = ps
    m1 = jnp.maximum(_dot(hn, m1h_ref[...]) + _dot(xs, m1x_ref[...]) + mb1_ref[...], 0.0)
    td_ref[:, _H:2 * _H] = _dot(m1, m2_ref[...]) + mb2_ref[...]
    td_ref[:, 2 * _H:3 * _H] = _dot(hc, wq_ref[...])
    ts_ref[:, 0:_H] = _dot(hn, gr_ref[...]) + _dot(xs, gxr_ref[...])
    ts_ref[:, _H:2 * _H] = _dot(hc, wk_ref[...])
    ts_ref[:, 2 * _H:3 * _H] = _dot(hc, wv_ref[...])


def _node_precompute(h, x_s, W_q, W_k, W_v, W_static, ln_ds_w, ln_ds_b,
                     msg_W1, msg_b1, msg_W2, msg_b2, gate_W1, gate_b1,
                     ln_w, ln_b):
    grid = _N // _BN
    row = lambda i: (i, 0)
    full = lambda i: (0, 0)
    hspec = pl.BlockSpec((_BN, _H), row)
    xspec = pl.BlockSpec((_BN, _S), row)
    wspec = pl.BlockSpec((_H, _H), full)
    sspec = pl.BlockSpec((_S, _H), full)
    bspec = pl.BlockSpec((1, _H), full)
    tspec = pl.BlockSpec((_BN, 3 * _H), row)
    out = [jax.ShapeDtypeStruct((_N, 3 * _H), jnp.float32),
           jax.ShapeDtypeStruct((_N, 3 * _H), jnp.float32),
           jax.ShapeDtypeStruct((_N, _H), jnp.float32)]
    return pl.pallas_call(
        _node_kernel,
        grid=(grid,),
        in_specs=[hspec, xspec, wspec, wspec, wspec, sspec, bspec, bspec,
                  wspec, sspec, bspec, wspec, bspec, wspec, wspec, sspec,
                  sspec, bspec, bspec, bspec],
        out_specs=[tspec, tspec, hspec],
        out_shape=out,
    )(h, x_s, W_q, W_k, W_v, W_static,
      ln_ds_w.reshape(1, _H), ln_ds_b.reshape(1, _H),
      msg_W1[:_H], msg_W1[_H:], msg_b1.reshape(1, _H), msg_W2,
      msg_b2.reshape(1, _H),
      gate_W1[:_H], gate_W1[_H:2 * _H], gate_W1[2 * _H:2 * _H + _S],
      gate_W1[2 * _H + _S:], gate_b1.reshape(1, _H),
      ln_w.reshape(1, _H), ln_b.reshape(1, _H))


_EC = 16


def _escore_kernel(qd_ref, ks_ref, sel_ref, e16_ref, e128_ref):
    prod = qd_ref[...] * ks_ref[...]
    s = _dot(prod, sel_ref[...]) * (1.0 / np.sqrt(float(_HD)))
    ex = jnp.exp(s)
    e16_ref[...] = ex[:, :_EC]
    e128_ref[...] = ex


def _escore(gd, gs):
    sel = jnp.repeat(jnp.eye(_NH, dtype=jnp.float32), _HD, axis=0)
    sel = jnp.concatenate(
        [sel, jnp.zeros((_H, 128 - _NH), jnp.float32)], axis=1)
    grid = _E // _BE
    return pl.pallas_call(
        _escore_kernel,
        grid=(grid,),
        in_specs=[pl.BlockSpec((_BE, _H), lambda i: (i, 2)),
                  pl.BlockSpec((_BE, _H), lambda i: (i, 1)),
                  pl.BlockSpec((_H, 128), lambda i: (0, 0))],
        out_specs=[pl.BlockSpec((_BE, _EC), lambda i: (i, 0)),
                   pl.BlockSpec((_BE, 128), lambda i: (i, 0))],
        out_shape=[jax.ShapeDtypeStruct((_E, _EC), jnp.float32),
                   jax.ShapeDtypeStruct((_E, 128), jnp.float32)],
    )(gd, gs, sel)


def _wvun_kernel(e_ref, vs_ref, exp_ref, wv_ref):
    wv_ref[...] = vs_ref[...] * _dot(e_ref[:, :_NH], exp_ref[...])


def _wv_un(e16, gs):
    expand = jnp.repeat(jnp.eye(_NH, dtype=jnp.float32), _HD, axis=1)
    grid = _E // _BE
    return pl.pallas_call(
        _wvun_kernel,
        grid=(grid,),
        in_specs=[pl.BlockSpec((_BE, _EC), lambda i: (i, 0)),
                  pl.BlockSpec((_BE, _H), lambda i: (i, 2)),
                  pl.BlockSpec((_NH, _H), lambda i: (0, 0))],
        out_specs=pl.BlockSpec((_BE, _H), lambda i: (i, 0)),
        out_shape=jax.ShapeDtypeStruct((_E, _H), jnp.float32),
    )(e16, gs, expand)


def _attn_kernel(e_ref, zd_ref, attn_ref):
    attn_ref[...] = e_ref[:, :_NH] / (zd_ref[:, :_NH] + 1e-9)


def _attn(e16, zd16):
    grid = _E // _BE
    row = lambda i: (i, 0)
    return pl.pallas_call(
        _attn_kernel,
        grid=(grid,),
        in_specs=[pl.BlockSpec((_BE, _EC), row), pl.BlockSpec((_BE, _EC), row)],
        out_specs=pl.BlockSpec((_BE, _NH), row),
        out_shape=jax.ShapeDtypeStruct((_E, _NH), jnp.float32),
    )(e16, zd16)


def _gate_kernel(xa_ref, xb_ref, m_ref, w2_ref, b2_ref, out_ref):
    g1 = jnp.maximum(xa_ref[...] + xb_ref[...], 0.0)
    g = jax.nn.sigmoid(_dot(g1, w2_ref[...]) + b2_ref[...])
    out_ref[...] = g * m_ref[...]


def _gate_mm(xa, ablk, xb, bblk, m, mblk, W2, b2, block):
    rows = xa.shape[0]
    grid = rows // block
    full = lambda i: (0, 0)
    return pl.pallas_call(
        _gate_kernel,
        grid=(grid,),
        in_specs=[pl.BlockSpec((block, _H), lambda i, a=ablk: (i, a)),
                  pl.BlockSpec((block, _H), lambda i, b=bblk: (i, b)),
                  pl.BlockSpec((block, _H), lambda i, c=mblk: (i, c)),
                  pl.BlockSpec((_H, _H), full), pl.BlockSpec((1, _H), full)],
        out_specs=pl.BlockSpec((block, _H), lambda i: (i, 0)),
        out_shape=jax.ShapeDtypeStruct((rows, _H), jnp.float32),
    )(xa, xb, m, W2, b2.reshape(1, _H))


_SC_BLK = 80
_SC_EPT = _E // 16
_SC_NBLK = _SC_EPT // _SC_BLK
_SC_ACC = _N + 8
_SC_ZPT = 624


def _sc_segsum_body(vals_hbm, idx_hbm, zeros_hbm, out_hbm,
                    idx0, idx1, valb0, valb1, acc, sems):
    c = lax.axis_index("c")
    s = lax.axis_index("s")
    col = c * 128
    ebase = s * _SC_EPT
    idxb = (idx0, idx1)
    valb = (valb0, valb1)

    @pl.when(s < 15)
    def _():
        pltpu.sync_copy(zeros_hbm.at[pl.ds(0, _SC_ZPT)],
                        acc.at[pl.ds(s * _SC_ZPT, _SC_ZPT)])

    @pl.when(s == 15)
    def _():
        pltpu.sync_copy(zeros_hbm,
                        acc.at[pl.ds(15 * _SC_ZPT, _SC_ACC - 15 * _SC_ZPT)])

    plsc.subcore_barrier()

    def _copies(g, slot):
        return [
            pltpu.make_async_copy(
                vals_hbm.at[pl.ds(ebase + g * _SC_BLK, _SC_BLK),
                            pl.ds(col, 128)],
                valb[slot], sems.at[slot]),
            pltpu.make_async_copy(
                idx_hbm.at[pl.ds(ebase + g * _SC_BLK, _SC_BLK)],
                idxb[slot], sems.at[slot]),
        ]

    def _start(g, slot):
        for cp in _copies(g, slot):
            cp.start()

    def _wait(g, slot):
        for cp in _copies(g, slot):
            cp.wait()

    def _scatter(slot):
        pltpu.sync_copy(valb[slot], acc.at[idxb[slot]], add=True)

    _start(0, 0)

    def _body(i, carry):
        g0 = 2 * i
        _wait(g0, 0)
        _start(g0 + 1, 1)
        _scatter(0)
        _wait(g0 + 1, 1)
        _start(g0 + 2, 0)
        _scatter(1)
        return carry

    lax.fori_loop(0, (_SC_NBLK - 1) // 2, _body, 0)
    _wait(_SC_NBLK - 1, 0)
    _scatter(0)

    plsc.subcore_barrier()

    @pl.when(s < 15)
    def _():
        pltpu.sync_copy(
            acc.at[pl.ds(s * _SC_ZPT, _SC_ZPT)],
            out_hbm.at[pl.ds(s * _SC_ZPT, _SC_ZPT), pl.ds(col, 128)])

    @pl.when(s == 15)
    def _():
        pltpu.sync_copy(
            acc.at[pl.ds(15 * _SC_ZPT, _N - 15 * _SC_ZPT)],
            out_hbm.at[pl.ds(15 * _SC_ZPT, _N - 15 * _SC_ZPT),
                       pl.ds(col, 128)])


_SCZ_BLK = 40
_SCZ_EPT = _E // 32
_SCZ_NBLK = _SCZ_EPT // _SCZ_BLK


def _sc_zsum_body(e_hbm, idx_hbm, zeros_hbm, out_hbm,
                  idx0, idx1, eb0, eb1, acc, sems):
    c = lax.axis_index("c")
    s = lax.axis_index("s")
    col = c * 128
    ebase = (c * 16 + s) * _SCZ_EPT
    idxb = (idx0, idx1)
    eb = (eb0, eb1)

    @pl.when(s < 15)
    def _():
        pltpu.sync_copy(zeros_hbm.at[pl.ds(0, _SC_ZPT)],
                        acc.at[pl.ds(s * _SC_ZPT, _SC_ZPT)])

    @pl.when(s == 15)
    def _():
        pltpu.sync_copy(zeros_hbm,
                        acc.at[pl.ds(15 * _SC_ZPT, _SC_ACC - 15 * _SC_ZPT)])

    plsc.subcore_barrier()

    def _copies(g, slot):
        return [
            pltpu.make_async_copy(
                e_hbm.at[pl.ds(ebase + g * _SCZ_BLK, _SCZ_BLK)],
                eb[slot], sems.at[slot]),
            pltpu.make_async_copy(
                idx_hbm.at[pl.ds(ebase + g * _SCZ_BLK, _SCZ_BLK)],
                idxb[slot], sems.at[slot]),
        ]

    def _start(g, slot):
        for cp in _copies(g, slot):
            cp.start()

    def _wait(g, slot):
        for cp in _copies(g, slot):
            cp.wait()

    def _scatter(slot):
        pltpu.sync_copy(eb[slot], acc.at[idxb[slot]], add=True)

    _start(0, 0)

    def _body(i, carry):
        g0 = 2 * i
        _wait(g0, 0)
        _start(g0 + 1, 1)
        _scatter(0)
        _wait(g0 + 1, 1)
        _start(g0 + 2, 0)
        _scatter(1)
        return carry

    lax.fori_loop(0, (_SCZ_NBLK - 1) // 2, _body, 0)
    _wait(_SCZ_NBLK - 1, 0)
    _scatter(0)

    plsc.subcore_barrier()

    @pl.when(s < 15)
    def _():
        pltpu.sync_copy(
            acc.at[pl.ds(s * _SC_ZPT, _SC_ZPT)],
            out_hbm.at[pl.ds(s * _SC_ZPT, _SC_ZPT), pl.ds(col, 128)])

    @pl.when(s == 15)
    def _():
        pltpu.sync_copy(
            acc.at[pl.ds(15 * _SC_ZPT, _N - 15 * _SC_ZPT)],
            out_hbm.at[pl.ds(15 * _SC_ZPT, _N - 15 * _SC_ZPT),
                       pl.ds(col, 128)])


def _sc_zsum(e128, idx):
    zeros = jnp.zeros((_SC_ACC - 15 * _SC_ZPT, 128), jnp.float32)
    mesh = plsc.VectorSubcoreMesh(core_axis_name="c", subcore_axis_name="s")
    f = pl.kernel(
        _sc_zsum_body,
        out_type=jax.ShapeDtypeStruct((_N, 256), jnp.float32),
        mesh=mesh,
        scratch_types=[
            pltpu.VMEM((_SCZ_BLK,), jnp.int32),
            pltpu.VMEM((_SCZ_BLK,), jnp.int32),
            pltpu.VMEM((_SCZ_BLK, 128), jnp.float32),
            pltpu.VMEM((_SCZ_BLK, 128), jnp.float32),
            pltpu.VMEM_SHARED((_SC_ACC, 128), jnp.float32),
            pltpu.SemaphoreType.DMA((2,)),
        ],
    )
    return f(e128, idx.astype(jnp.int32), zeros)


def _sc_segsum(vals, idx):
    zeros = jnp.zeros((_SC_ACC - 15 * _SC_ZPT, 128), jnp.float32)
    mesh = plsc.VectorSubcoreMesh(core_axis_name="c", subcore_axis_name="s")
    f = pl.kernel(
        _sc_segsum_body,
        out_type=jax.ShapeDtypeStruct((_N, _H), jnp.float32),
        mesh=mesh,
        scratch_types=[
            pltpu.VMEM((_SC_BLK,), jnp.int32),
            pltpu.VMEM((_SC_BLK,), jnp.int32),
            pltpu.VMEM((_SC_BLK, 128), jnp.float32),
            pltpu.VMEM((_SC_BLK, 128), jnp.float32),
            pltpu.VMEM_SHARED((_SC_ACC, 128), jnp.float32),
            pltpu.SemaphoreType.DMA((2,)),
        ],
    )
    return f(vals, idx.astype(jnp.int32), zeros)


_SCW_EPT = _E // 32
_SCW_VREGS = _SCW_EPT // 16
_SCW_TAIL = _SCW_EPT - 16 * _SCW_VREGS


def _sc_winner_body(idx_hbm, out_hbm, ibuf, tbl, sem):
    c = lax.axis_index("c")
    s = lax.axis_index("s")
    w = c * 16 + s
    base = w * _SCW_EPT

    ibuf[pl.ds(_SCW_EPT - _SCW_TAIL, 16)] = jnp.zeros((16,), jnp.int32)
    cp = pltpu.make_async_copy(idx_hbm.at[pl.ds(base, _SCW_EPT)],
                               ibuf.at[pl.ds(0, _SCW_EPT)], sem)
    cp.start()

    def _init(i, carry):
        tbl[pl.ds(i * 16, 16)] = jnp.full((16,), -1, jnp.int32)
        return carry

    lax.fori_loop(0, _N // 16, _init, 0)
    cp.wait()

    lanes = lax.iota(jnp.int32, 16)
    masks = [lanes == l for l in range(16)]

    def _step(j, carry):
        iv = ibuf[pl.ds(j * 16, 16)]
        ids = lanes + (base + j * 16)
        for l in range(16):
            plsc.store_scatter(tbl, (iv,), ids, mask=masks[l])
        return carry

    lax.fori_loop(0, _SCW_VREGS, _step, 0)
    ivt = ibuf[pl.ds(_SCW_VREGS * 16, 16)]
    idt = lanes + (base + _SCW_VREGS * 16)
    valid = lanes < _SCW_TAIL
    for l in range(_SCW_TAIL):
        plsc.store_scatter(tbl, (ivt,), idt, mask=masks[l] & valid)

    pltpu.sync_copy(tbl, out_hbm.at[pl.ds(w * _N, _N)])


def _sc_winner(idx):
    mesh = plsc.VectorSubcoreMesh(core_axis_name="c", subcore_axis_name="s")
    f = pl.kernel(
        _sc_winner_body,
        out_type=jax.ShapeDtypeStruct((32 * _N,), jnp.int32),
        mesh=mesh,
        scratch_types=[
            pltpu.VMEM((_SCW_EPT + 16,), jnp.int32),
            pltpu.VMEM((_N,), jnp.int32),
            pltpu.SemaphoreType.DMA,
        ],
    )
    return f(idx.astype(jnp.int32))


def _combine_kernel(h_ref, ag_ref, z_ref, up_ref, exp_ref, wout_ref,
                    wc1_ref, wc2_ref, bc_ref, out_ref):
    recip = 1.0 / (z_ref[:, :_NH] + 1e-9)
    ag = ag_ref[...] * _dot(recip, exp_ref[...])
    ds = _dot(ag, wout_ref[...])
    out_ref[...] = (h_ref[...] + _dot(ds, wc1_ref[...])
                    + _dot(up_ref[...], wc2_ref[...]) + bc_ref[...])


def _combine(h, aggr_un, z16, upstream, W_out, W_comb, b_comb):
    expand = jnp.repeat(jnp.eye(_NH, dtype=jnp.float32), _HD, axis=1)
    grid = _N // _BN
    row = lambda i: (i, 0)
    full = lambda i: (0, 0)
    return pl.pallas_call(
        _combine_kernel,
        grid=(grid,),
        in_specs=[pl.BlockSpec((_BN, _H), row), pl.BlockSpec((_BN, _H), row),
                  pl.BlockSpec((_BN, _EC), row), pl.BlockSpec((_BN, _H), row),
                  pl.BlockSpec((_NH, _H), full)]
        + [pl.BlockSpec((_H, _H), full)] * 3
        + [pl.BlockSpec((1, _H), full)],
        out_specs=pl.BlockSpec((_BN, _H), row),
        out_shape=jax.ShapeDtypeStruct((_N, _H), jnp.float32),
    )(h, aggr_un, z16, upstream, expand, W_out, W_comb[:_H], W_comb[_H:],
      b_comb.reshape(1, _H))


def kernel(h, x_s, edge_index, node_mask, edge_mask, W_q, W_k, W_v, W_static, W_out, ln_ds_w, ln_ds_b, msg_W1, msg_b1, msg_W2, msg_b2, gate_W1, gate_b1, gate_W2, gate_b2, ln_w, ln_b, W_comb, b_comb):
    src = edge_index[0]
    dest = edge_index[1]
    td, ts, ps = _node_precompute(
        h, x_s, W_q, W_k, W_v, W_static, ln_ds_w, ln_ds_b,
        msg_W1, msg_b1, msg_W2, msg_b2, gate_W1, gate_b1, ln_w, ln_b)

    gd = td[dest]
    gs = ts[src]
    e16, e128 = _escore(gd, gs)
    wv = _wv_un(e16, gs)
    aggr_un = _sc_segsum(wv, dest)
    zpart = _sc_zsum(e128, dest)
    z16 = zpart[:, :_EC] + zpart[:, 128:128 + _EC]
    zd = z16[dest]
    attn = _attn(e16, zd)

    gated = _gate_mm(gd, 0, gs, 0, gd, 1, gate_W2, gate_b2, _BE)
    upstream = _sc_segsum(gated, src)

    winner = jnp.full((_N,), -1, jnp.int32).at[src].max(
        jnp.arange(_E, dtype=jnp.int32))
    has = winner >= 0
    partner = dest[jnp.where(has, winner, 0)]
    psp = ps[partner]
    hasf = jnp.broadcast_to(has[:, None], (_N, _H)).astype(jnp.float32)
    gpn = _gate_mm(psp, 0, ts, 0, hasf, 0, gate_W2, gate_b2, _BN)

    out = _combine(h, aggr_un, z16, upstream, W_out, W_comb, b_comb)
    return (out, attn, gpn)


# speedup vs baseline: 3.1674x; 1.0195x over previous
import functools

import jax
import jax.numpy as jnp
import numpy as np
from jax import lax
from jax.experimental import pallas as pl
from jax.experimental.pallas import tpu as pltpu
from jax.experimental.pallas import tpu_sc as plsc

_N = 10000
_E = 160000
_H = 256
_NH = 8
_HD = 32
_S = 16

_BN = 1000
_BE = 4000


def _lnorm(x, w, b):
    m = jnp.mean(x, axis=-1, keepdims=True)
    v = jnp.mean((x - m) ** 2, axis=-1, keepdims=True)
    return (x - m) / jnp.sqrt(v + 1e-5) * w + b


def _dot(a, b):
    return jnp.dot(a, b, preferred_element_type=jnp.float32)


def _node_kernel(h_ref, xs_ref, wq_ref, wk_ref, wv_ref, wstat_ref,
                 lnds_w_ref, lnds_b_ref, m1h_ref, m1x_ref, mb1_ref,
                 m2_ref, mb2_ref, gs_ref, gr_ref, gxs_ref, gxr_ref,
                 gb1_ref, lnw_ref, lnb_ref,
                 td_ref, ts_ref, ps_ref):
    h = h_ref[...]
    xs = xs_ref[...]
    hn = _lnorm(h, lnw_ref[...], lnb_ref[...])
    hc = _lnorm(hn + _dot(xs, wstat_ref[...]), lnds_w_ref[...], lnds_b_ref[...])
    ps = _dot(hn, gs_ref[...]) + _dot(xs, gxs_ref[...]) + gb1_ref[...]
    td_ref[:, 0:_H] = ps
    ps_ref[...] = ps
    m1 = jnp.maximum(_dot(hn, m1h_ref[...]) + _dot(xs, m1x_ref[...]) + mb1_ref[...], 0.0)
    td_ref[:, _H:2 * _H] = _dot(m1, m2_ref[...]) + mb2_ref[...]
    td_ref[:, 2 * _H:3 * _H] = _dot(hc, wq_ref[...])
    ts_ref[:, 0:_H] = _dot(hn, gr_ref[...]) + _dot(xs, gxr_ref[...])
    ts_ref[:, _H:2 * _H] = _dot(hc, wk_ref[...])
    ts_ref[:, 2 * _H:3 * _H] = _dot(hc, wv_ref[...])


def _node_precompute(h, x_s, W_q, W_k, W_v, W_static, ln_ds_w, ln_ds_b,
                     msg_W1, msg_b1, msg_W2, msg_b2, gate_W1, gate_b1,
                     ln_w, ln_b):
    grid = _N // _BN
    row = lambda i: (i, 0)
    full = lambda i: (0, 0)
    hspec = pl.BlockSpec((_BN, _H), row)
    xspec = pl.BlockSpec((_BN, _S), row)
    wspec = pl.BlockSpec((_H, _H), full)
    sspec = pl.BlockSpec((_S, _H), full)
    bspec = pl.BlockSpec((1, _H), full)
    tspec = pl.BlockSpec((_BN, 3 * _H), row)
    out = [jax.ShapeDtypeStruct((_N, 3 * _H), jnp.float32),
           jax.ShapeDtypeStruct((_N, 3 * _H), jnp.float32),
           jax.ShapeDtypeStruct((_N, _H), jnp.float32)]
    return pl.pallas_call(
        _node_kernel,
        grid=(grid,),
        in_specs=[hspec, xspec, wspec, wspec, wspec, sspec, bspec, bspec,
                  wspec, sspec, bspec, wspec, bspec, wspec, wspec, sspec,
                  sspec, bspec, bspec, bspec],
        out_specs=[tspec, tspec, hspec],
        out_shape=out,
    )(h, x_s, W_q, W_k, W_v, W_static,
      ln_ds_w.reshape(1, _H), ln_ds_b.reshape(1, _H),
      msg_W1[:_H], msg_W1[_H:], msg_b1.reshape(1, _H), msg_W2,
      msg_b2.reshape(1, _H),
      gate_W1[:_H], gate_W1[_H:2 * _H], gate_W1[2 * _H:2 * _H + _S],
      gate_W1[2 * _H + _S:], gate_b1.reshape(1, _H),
      ln_w.reshape(1, _H), ln_b.reshape(1, _H))


_EC = 16


def _escore_kernel(qd_ref, ks_ref, sel_ref, e16_ref, e128_ref):
    prod = qd_ref[...] * ks_ref[...]
    s = _dot(prod, sel_ref[...]) * (1.0 / np.sqrt(float(_HD)))
    ex = jnp.exp(s)
    e16_ref[...] = ex[:, :_EC]
    e128_ref[...] = ex


def _escore(gd, gs):
    sel = jnp.repeat(jnp.eye(_NH, dtype=jnp.float32), _HD, axis=0)
    sel = jnp.concatenate(
        [sel, jnp.zeros((_H, 128 - _NH), jnp.float32)], axis=1)
    grid = _E // _BE
    return pl.pallas_call(
        _escore_kernel,
        grid=(grid,),
        in_specs=[pl.BlockSpec((_BE, _H), lambda i: (i, 2)),
                  pl.BlockSpec((_BE, _H), lambda i: (i, 1)),
                  pl.BlockSpec((_H, 128), lambda i: (0, 0))],
        out_specs=[pl.BlockSpec((_BE, _EC), lambda i: (i, 0)),
                   pl.BlockSpec((_BE, 128), lambda i: (i, 0))],
        out_shape=[jax.ShapeDtypeStruct((_E, _EC), jnp.float32),
                   jax.ShapeDtypeStruct((_E, 128), jnp.float32)],
    )(gd, gs, sel)


def _wvun_kernel(e_ref, vs_ref, exp_ref, wv_ref):
    wv_ref[...] = vs_ref[...] * _dot(e_ref[:, :_NH], exp_ref[...])


def _wv_un(e16, gs):
    expand = jnp.repeat(jnp.eye(_NH, dtype=jnp.float32), _HD, axis=1)
    grid = _E // _BE
    return pl.pallas_call(
        _wvun_kernel,
        grid=(grid,),
        in_specs=[pl.BlockSpec((_BE, _EC), lambda i: (i, 0)),
                  pl.BlockSpec((_BE, _H), lambda i: (i, 2)),
                  pl.BlockSpec((_NH, _H), lambda i: (0, 0))],
        out_specs=pl.BlockSpec((_BE, _H), lambda i: (i, 0)),
        out_shape=jax.ShapeDtypeStruct((_E, _H), jnp.float32),
    )(e16, gs, expand)


def _attn_kernel(e_ref, zd_ref, attn_ref):
    attn_ref[...] = e_ref[:, :_NH] / (zd_ref[:, :_NH] + 1e-9)


def _attn(e16, zd16):
    grid = _E // _BE
    row = lambda i: (i, 0)
    return pl.pallas_call(
        _attn_kernel,
        grid=(grid,),
        in_specs=[pl.BlockSpec((_BE, _EC), row), pl.BlockSpec((_BE, _EC), row)],
        out_specs=pl.BlockSpec((_BE, _NH), row),
        out_shape=jax.ShapeDtypeStruct((_E, _NH), jnp.float32),
    )(e16, zd16)


def _gate_kernel(xa_ref, xb_ref, m_ref, w2_ref, b2_ref, out_ref):
    g1 = jnp.maximum(xa_ref[...] + xb_ref[...], 0.0)
    g = jax.nn.sigmoid(_dot(g1, w2_ref[...]) + b2_ref[...])
    out_ref[...] = g * m_ref[...]


def _gate_mm(xa, ablk, xb, bblk, m, mblk, W2, b2, block):
    rows = xa.shape[0]
    grid = rows // block
    full = lambda i: (0, 0)
    return pl.pallas_call(
        _gate_kernel,
        grid=(grid,),
        in_specs=[pl.BlockSpec((block, _H), lambda i, a=ablk: (i, a)),
                  pl.BlockSpec((block, _H), lambda i, b=bblk: (i, b)),
                  pl.BlockSpec((block, _H), lambda i, c=mblk: (i, c)),
                  pl.BlockSpec((_H, _H), full), pl.BlockSpec((1, _H), full)],
        out_specs=pl.BlockSpec((block, _H), lambda i: (i, 0)),
        out_shape=jax.ShapeDtypeStruct((rows, _H), jnp.float32),
    )(xa, xb, m, W2, b2.reshape(1, _H))


_SC_BLK = 80
_SC_EPT = _E // 16
_SC_NBLK = _SC_EPT // _SC_BLK
_SC_ACC = _N + 8
_SC_ZPT = 624


def _sc_segsum_body(vals_hbm, idx_hbm, zeros_hbm, out_hbm,
                    idx0, idx1, valb0, valb1, acc, sems):
    c = lax.axis_index("c")
    s = lax.axis_index("s")
    col = c * 128
    ebase = s * _SC_EPT
    idxb = (idx0, idx1)
    valb = (valb0, valb1)

    @pl.when(s < 15)
    def _():
        pltpu.sync_copy(zeros_hbm.at[pl.ds(0, _SC_ZPT)],
                        acc.at[pl.ds(s * _SC_ZPT, _SC_ZPT)])

    @pl.when(s == 15)
    def _():
        pltpu.sync_copy(zeros_hbm,
                        acc.at[pl.ds(15 * _SC_ZPT, _SC_ACC - 15 * _SC_ZPT)])

    plsc.subcore_barrier()

    def _copies(g, slot):
        return [
            pltpu.make_async_copy(
                vals_hbm.at[pl.ds(ebase + g * _SC_BLK, _SC_BLK),
                            pl.ds(col, 128)],
                valb[slot], sems.at[slot]),
            pltpu.make_async_copy(
                idx_hbm.at[pl.ds(ebase + g * _SC_BLK, _SC_BLK)],
                idxb[slot], sems.at[slot]),
        ]

    def _start(g, slot):
        for cp in _copies(g, slot):
            cp.start()

    def _wait(g, slot):
        for cp in _copies(g, slot):
            cp.wait()

    def _scatter(slot):
        pltpu.sync_copy(valb[slot], acc.at[idxb[slot]], add=True)

    _start(0, 0)

    def _body(i, carry):
        g0 = 2 * i
        _wait(g0, 0)
        _start(g0 + 1, 1)
        _scatter(0)
        _wait(g0 + 1, 1)
        _start(g0 + 2, 0)
        _scatter(1)
        return carry

    lax.fori_loop(0, (_SC_NBLK - 1) // 2, _body, 0)
    _wait(_SC_NBLK - 1, 0)
    _scatter(0)

    plsc.subcore_barrier()

    @pl.when(s < 15)
    def _():
        pltpu.sync_copy(
            acc.at[pl.ds(s * _SC_ZPT, _SC_ZPT)],
            out_hbm.at[pl.ds(s * _SC_ZPT, _SC_ZPT), pl.ds(col, 128)])

    @pl.when(s == 15)
    def _():
        pltpu.sync_copy(
            acc.at[pl.ds(15 * _SC_ZPT, _N - 15 * _SC_ZPT)],
            out_hbm.at[pl.ds(15 * _SC_ZPT, _N - 15 * _SC_ZPT),
                       pl.ds(col, 128)])


_SCZ_BLK = 40
_SCZ_EPT = _E // 32
_SCZ_NBLK = _SCZ_EPT // _SCZ_BLK


def _sc_zsum_body(e_hbm, idx_hbm, zeros_hbm, out_hbm,
                  idx0, idx1, eb0, eb1, acc, sems):
    c = lax.axis_index("c")
    s = lax.axis_index("s")
    col = c * 128
    ebase = (c * 16 + s) * _SCZ_EPT
    idxb = (idx0, idx1)
    eb = (eb0, eb1)

    @pl.when(s < 15)
    def _():
        pltpu.sync_copy(zeros_hbm.at[pl.ds(0, _SC_ZPT)],
                        acc.at[pl.ds(s * _SC_ZPT, _SC_ZPT)])

    @pl.when(s == 15)
    def _():
        pltpu.sync_copy(zeros_hbm,
                        acc.at[pl.ds(15 * _SC_ZPT, _SC_ACC - 15 * _SC_ZPT)])

    plsc.subcore_barrier()

    def _copies(g, slot):
        return [
            pltpu.make_async_copy(
                e_hbm.at[pl.ds(ebase + g * _SCZ_BLK, _SCZ_BLK)],
                eb[slot], sems.at[slot]),
            pltpu.make_async_copy(
                idx_hbm.at[pl.ds(ebase + g * _SCZ_BLK, _SCZ_BLK)],
                idxb[slot], sems.at[slot]),
        ]

    def _start(g, slot):
        for cp in _copies(g, slot):
            cp.start()

    def _wait(g, slot):
        for cp in _copies(g, slot):
            cp.wait()

    def _scatter(slot):
        pltpu.sync_copy(eb[slot], acc.at[idxb[slot]], add=True)

    _start(0, 0)

    def _body(i, carry):
        g0 = 2 * i
        _wait(g0, 0)
        _start(g0 + 1, 1)
        _scatter(0)
        _wait(g0 + 1, 1)
        _start(g0 + 2, 0)
        _scatter(1)
        return carry

    lax.fori_loop(0, (_SCZ_NBLK - 1) // 2, _body, 0)
    _wait(_SCZ_NBLK - 1, 0)
    _scatter(0)

    plsc.subcore_barrier()

    @pl.when(s < 15)
    def _():
        pltpu.sync_copy(
            acc.at[pl.ds(s * _SC_ZPT, _SC_ZPT)],
            out_hbm.at[pl.ds(s * _SC_ZPT, _SC_ZPT), pl.ds(col, 128)])

    @pl.when(s == 15)
    def _():
        pltpu.sync_copy(
            acc.at[pl.ds(15 * _SC_ZPT, _N - 15 * _SC_ZPT)],
            out_hbm.at[pl.ds(15 * _SC_ZPT, _N - 15 * _SC_ZPT),
                       pl.ds(col, 128)])


def _sc_zsum(e128, idx):
    zeros = jnp.zeros((_SC_ACC - 15 * _SC_ZPT, 128), jnp.float32)
    mesh = plsc.VectorSubcoreMesh(core_axis_name="c", subcore_axis_name="s")
    f = pl.kernel(
        _sc_zsum_body,
        out_type=jax.ShapeDtypeStruct((_N, 256), jnp.float32),
        mesh=mesh,
        scratch_types=[
            pltpu.VMEM((_SCZ_BLK,), jnp.int32),
            pltpu.VMEM((_SCZ_BLK,), jnp.int32),
            pltpu.VMEM((_SCZ_BLK, 128), jnp.float32),
            pltpu.VMEM((_SCZ_BLK, 128), jnp.float32),
            pltpu.VMEM_SHARED((_SC_ACC, 128), jnp.float32),
            pltpu.SemaphoreType.DMA((2,)),
        ],
    )
    return f(e128, idx.astype(jnp.int32), zeros)


def _sc_segsum(vals, idx):
    zeros = jnp.zeros((_SC_ACC - 15 * _SC_ZPT, 128), jnp.float32)
    mesh = plsc.VectorSubcoreMesh(core_axis_name="c", subcore_axis_name="s")
    f = pl.kernel(
        _sc_segsum_body,
        out_type=jax.ShapeDtypeStruct((_N, _H), jnp.float32),
        mesh=mesh,
        scratch_types=[
            pltpu.VMEM((_SC_BLK,), jnp.int32),
            pltpu.VMEM((_SC_BLK,), jnp.int32),
            pltpu.VMEM((_SC_BLK, 128), jnp.float32),
            pltpu.VMEM((_SC_BLK, 128), jnp.float32),
            pltpu.VMEM_SHARED((_SC_ACC, 128), jnp.float32),
            pltpu.SemaphoreType.DMA((2,)),
        ],
    )
    return f(vals, idx.astype(jnp.int32), zeros)


_SCW_EPT = _E // 32
_SCW_VREGS = _SCW_EPT // 16
_SCW_TAIL = _SCW_EPT - 16 * _SCW_VREGS


def _sc_winner_body(idx_hbm, out_hbm, ibuf, tbl, sem):
    c = lax.axis_index("c")
    s = lax.axis_index("s")
    w = c * 16 + s
    base = w * _SCW_EPT

    ibuf[pl.ds(_SCW_EPT - _SCW_TAIL, 16)] = jnp.zeros((16,), jnp.int32)
    cp = pltpu.make_async_copy(idx_hbm.at[pl.ds(base, _SCW_EPT)],
                               ibuf.at[pl.ds(0, _SCW_EPT)], sem)
    cp.start()

    def _init(i, carry):
        tbl[pl.ds(i * 16, 16)] = jnp.full((16,), -1, jnp.int32)
        return carry

    lax.fori_loop(0, _N // 16, _init, 0)
    cp.wait()

    lanes = lax.iota(jnp.int32, 16)
    masks = [lanes == l for l in range(16)]

    def _step(j, carry):
        iv = ibuf[pl.ds(j * 16, 16)]
        ids = lanes + (base + j * 16)
        for l in range(16):
            plsc.store_scatter(tbl, (iv,), ids, mask=masks[l])
        return carry

    lax.fori_loop(0, _SCW_VREGS, _step, 0)
    ivt = ibuf[pl.ds(_SCW_VREGS * 16, 16)]
    idt = lanes + (base + _SCW_VREGS * 16)
    valid = lanes < _SCW_TAIL
    for l in range(_SCW_TAIL):
        plsc.store_scatter(tbl, (ivt,), idt, mask=masks[l] & valid)

    pltpu.sync_copy(tbl, out_hbm.at[pl.ds(w * _N, _N)])


def _sc_winner(idx):
    mesh = plsc.VectorSubcoreMesh(core_axis_name="c", subcore_axis_name="s")
    f = pl.kernel(
        _sc_winner_body,
        out_type=jax.ShapeDtypeStruct((32 * _N,), jnp.int32),
        mesh=mesh,
        scratch_types=[
            pltpu.VMEM((_SCW_EPT + 16,), jnp.int32),
            pltpu.VMEM((_N,), jnp.int32),
            pltpu.SemaphoreType.DMA,
        ],
    )
    return f(idx.astype(jnp.int32))


def _combine_kernel(h_ref, ag_ref, z_ref, up_ref, exp_ref, wout_ref,
                    wc1_ref, wc2_ref, bc_ref, out_ref):
    recip = 1.0 / (z_ref[:, :_NH] + 1e-9)
    ag = ag_ref[...] * _dot(recip, exp_ref[...])
    ds = _dot(ag, wout_ref[...])
    out_ref[...] = (h_ref[...] + _dot(ds, wc1_ref[...])
                    + _dot(up_ref[...], wc2_ref[...]) + bc_ref[...])


def _combine(h, aggr_un, z16, upstream, W_out, W_comb, b_comb):
    expand = jnp.repeat(jnp.eye(_NH, dtype=jnp.float32), _HD, axis=1)
    grid = _N // _BN
    row = lambda i: (i, 0)
    full = lambda i: (0, 0)
    return pl.pallas_call(
        _combine_kernel,
        grid=(grid,),
        in_specs=[pl.BlockSpec((_BN, _H), row), pl.BlockSpec((_BN, _H), row),
                  pl.BlockSpec((_BN, _EC), row), pl.BlockSpec((_BN, _H), row),
                  pl.BlockSpec((_NH, _H), full)]
        + [pl.BlockSpec((_H, _H), full)] * 3
        + [pl.BlockSpec((1, _H), full)],
        out_specs=pl.BlockSpec((_BN, _H), row),
        out_shape=jax.ShapeDtypeStruct((_N, _H), jnp.float32),
    )(h, aggr_un, z16, upstream, expand, W_out, W_comb[:_H], W_comb[_H:],
      b_comb.reshape(1, _H))


def kernel(h, x_s, edge_index, node_mask, edge_mask, W_q, W_k, W_v, W_static, W_out, ln_ds_w, ln_ds_b, msg_W1, msg_b1, msg_W2, msg_b2, gate_W1, gate_b1, gate_W2, gate_b2, ln_w, ln_b, W_comb, b_comb):
    src = edge_index[0]
    dest = edge_index[1]
    td, ts, ps = _node_precompute(
        h, x_s, W_q, W_k, W_v, W_static, ln_ds_w, ln_ds_b,
        msg_W1, msg_b1, msg_W2, msg_b2, gate_W1, gate_b1, ln_w, ln_b)

    gd = td[dest]
    gs = ts[src]
    e16, e128 = _escore(gd, gs)
    wv = _wv_un(e16, gs)
    aggr_un = _sc_segsum(wv, dest)
    zpart = _sc_zsum(e128, dest)
    z16 = zpart[:, :_EC] + zpart[:, 128:128 + _EC]
    zd = z16[dest]
    attn = _attn(e16, zd)

    gated = _gate_mm(gd, 0, gs, 0, gd, 1, gate_W2, gate_b2, _BE)
    upstream = _sc_segsum(gated, src)

    winner = jnp.full((_N,), -1, jnp.int32).at[src].max(
        jnp.arange(_E, dtype=jnp.int32))
    has = winner >= 0
    partner = dest[jnp.where(has, winner, 0)]
    psp = ps[partner]
    hasf = jnp.broadcast_to(has[:, None], (_N, _H)).astype(jnp.float32)
    gpn = _gate_mm(psp, 0, ts, 0, hasf, 0, gate_W2, gate_b2, _BN)

    out = _combine(h, aggr_un, z16, upstream, W_out, W_comb, b_comb)
    return (out, attn, gpn)
